# baseline (speedup 1.0000x reference)
import functools
import jax
import jax.numpy as jnp
from jax.experimental import pallas as pl
from jax.experimental.pallas import tpu as pltpu

N_ROWS_BLK = 1000


def _lrelu(t):
    return jnp.where(t >= 0, t, 0.01 * t)


def _combine_body(h_ref, w_ref, b_ref, xh_ref, o_ref):
    t = jnp.dot(h_ref[...], w_ref[...], preferred_element_type=jnp.float32)
    o_ref[...] = _lrelu(t + b_ref[...] + xh_ref[...])


def _combine(h, w, b, xh):
    n, d = h.shape
    grid = n // N_ROWS_BLK
    return pl.pallas_call(
        _combine_body,
        grid=(grid,),
        in_specs=[
            pl.BlockSpec((N_ROWS_BLK, d), lambda i: (i, 0)),
            pl.BlockSpec((d, d), lambda i: (0, 0)),
            pl.BlockSpec((1, d), lambda i: (0, 0)),
            pl.BlockSpec((N_ROWS_BLK, d), lambda i: (i, 0)),
        ],
        out_specs=pl.BlockSpec((N_ROWS_BLK, d), lambda i: (i, 0)),
        out_shape=jax.ShapeDtypeStruct((n, d), jnp.float32),
    )(h, w, b.reshape(1, d), xh)


def _l2norm(x):
    n = jnp.sqrt(jnp.sum(x * x, axis=-1, keepdims=True))
    return x / jnp.maximum(n, 1e-12)


def _graph_gat(x, edge_index, weight):
    n = x.shape[0]
    x = x @ weight
    src = edge_index[0]
    dst = edge_index[1]
    mask = src != dst
    mf = mask.astype(x.dtype)
    x_j = jnp.take(x, src, axis=0)
    x_i = jnp.take(x, dst, axis=0)
    inner = jnp.sum(x_i * x_j, axis=-1)
    deg = jnp.zeros((n,), x.dtype).at[src].add(mf)
    deg_inv_sqrt = jnp.where(deg > 0, deg, 1.0) ** -0.5
    gate = jax.nn.sigmoid(jnp.take(deg_inv_sqrt, src) * inner)
    logits = jnp.where(mask, inner * gate, -1e9)
    seg_max = jax.ops.segment_max(logits, dst, num_segments=n)
    seg_max = jnp.where(jnp.isfinite(seg_max), seg_max, 0.0)
    ex = jnp.exp(logits - jnp.take(seg_max, dst)) * mf
    denom = jax.ops.segment_sum(ex, dst, num_segments=n)
    att = ex / (jnp.take(denom, dst) + 1e-16)
    out = jax.ops.segment_sum(x_j * att[:, None], dst, num_segments=n)
    return _l2norm(out)


def kernel(id_embedding, edge_index, features, preference, mlp_w, mlp_b, conv1_w,
           lin1_w, lin1_b, g1_w, g1_b, conv2_w, lin2_w, lin2_b, g2_w, g2_b):
    temp_features = jnp.tanh(features @ mlp_w + mlp_b)
    x = jnp.concatenate([preference, temp_features], axis=0)
    x = _l2norm(x)
    h = _lrelu(_graph_gat(x, edge_index, conv1_w))
    x_hat = _lrelu(x @ lin1_w + lin1_b) + id_embedding
    x1 = _combine(h, g1_w, g1_b, x_hat)
    h = _lrelu(_graph_gat(x1, edge_index, conv2_w))
    x_hat = _lrelu(x1 @ lin2_w + lin2_b) + id_embedding
    x2 = _combine(h, g2_w, g2_b, x_hat)
    return jnp.concatenate([x1, x2], axis=1)


# speedup vs baseline: 2.4240x; 2.4240x over previous
import functools
import jax
import jax.numpy as jnp
from jax import lax
from jax.experimental import pallas as pl
from jax.experimental.pallas import tpu as pltpu
from jax.experimental.pallas import tpu_sc as plsc

N_ROWS_BLK = 1000

_NC = 2
_NS = 16
_NW = _NC * _NS
_L = 16

_N = 50000
_NP = 51200
_E = 800000
_EP = 802816
_TE = _EP // _NW
_K = 512
_NB = _TE // _K
_G = _K // _L

_sc_mesh = plsc.VectorSubcoreMesh(
    core_axis_name="c", subcore_axis_name="s", num_cores=_NC, num_subcores=_NS)


def _wid():
    return lax.axis_index("s") * _NC + lax.axis_index("c")


def _deg_body(src_hbm, dst_hbm, zeros_hbm, out_hbm, sv, dv, acc, sem):
    wid = _wid()
    pltpu.sync_copy(zeros_hbm, acc)
    base = wid * _TE

    def blk(b, _):
        off = base + b * _K
        pltpu.sync_copy(src_hbm.at[pl.ds(off, _K)], sv)
        pltpu.sync_copy(dst_hbm.at[pl.ds(off, _K)], dv)

        def grp(g, _):
            s16 = sv[pl.ds(g * _L, _L)]
            d16 = dv[pl.ds(g * _L, _L)]
            m = s16 != d16
            plsc.addupdate_scatter(acc, [s16], jnp.ones((_L,), jnp.float32), mask=m)
            return 0

        lax.fori_loop(0, _G, grp, 0)
        return 0

    lax.fori_loop(0, _NB, blk, 0)
    pltpu.sync_copy(acc, out_hbm.at[wid])


_deg_call = pl.kernel(
    _deg_body,
    out_type=jax.ShapeDtypeStruct((_NW, _NP), jnp.float32),
    mesh=_sc_mesh,
    scratch_types=[
        pltpu.VMEM((_K,), jnp.int32),
        pltpu.VMEM((_K,), jnp.int32),
        pltpu.VMEM((_NP,), jnp.float32),
        pltpu.SemaphoreType.DMA,
    ],
    compiler_params=pltpu.CompilerParams(needs_layout_passes=False, use_tc_tiling_on_sc=False),
)


def _edge_body(src_hbm, dst_hbm, xw_hbm, dis_hbm, barr_hbm, zeros_hbm,
               ex_hbm, dpart_hbm, sv, dv, xjv, xiv, disv, bv, exb, dacc, sem):
    wid = _wid()
    pltpu.sync_copy(zeros_hbm, dacc)
    base = wid * _TE
    lanes = lax.iota(jnp.int32, _L)

    def blk(b, _):
        off = base + b * _K
        pltpu.sync_copy(src_hbm.at[pl.ds(off, _K)], sv)
        pltpu.sync_copy(dst_hbm.at[pl.ds(off, _K)], dv)
        c1 = pltpu.async_copy(xw_hbm.at[sv], xjv, sem)
        c2 = pltpu.async_copy(xw_hbm.at[dv], xiv, sem)
        c3 = pltpu.async_copy(dis_hbm.at[sv], disv, sem)
        c4 = pltpu.async_copy(barr_hbm.at[dv], bv, sem)
        c1.wait(); c2.wait(); c3.wait(); c4.wait()

        def grp(g, _):
            e0 = g * _L
            el = lanes + e0
            s16 = sv[pl.ds(e0, _L)]
            d16 = dv[pl.ds(e0, _L)]
            m = s16 != d16
            acc = jnp.zeros((_L,), jnp.float32)
            for dcol in range(64):
                cc = jnp.full((_L,), dcol, jnp.int32)
                a = plsc.load_gather(xiv, [el, cc])
                bb = plsc.load_gather(xjv, [el, cc])
                acc = acc + a * bb
            dis16 = disv[pl.ds(e0, _L)]
            b16 = bv[pl.ds(e0, _L)]
            gate = 1.0 / (1.0 + jnp.exp(-dis16 * acc))
            ex16 = jnp.exp(acc * gate - b16)
            ex16 = jnp.where(m, ex16, 0.0)
            exb[pl.ds(e0, _L)] = ex16
            plsc.addupdate_scatter(dacc, [d16], ex16, mask=m)
            return 0

        lax.fori_loop(0, _G, grp, 0)
        pltpu.sync_copy(exb, ex_hbm.at[pl.ds(off, _K)])
        return 0

    lax.fori_loop(0, _NB, blk, 0)
    pltpu.sync_copy(dacc, dpart_hbm.at[wid])


_edge_call = pl.kernel(
    _edge_body,
    out_type=(jax.ShapeDtypeStruct((_EP,), jnp.float32),
              jax.ShapeDtypeStruct((_NW, _NP), jnp.float32)),
    mesh=_sc_mesh,
    scratch_types=[
        pltpu.VMEM((_K,), jnp.int32),
        pltpu.VMEM((_K,), jnp.int32),
        pltpu.VMEM((_K, 64), jnp.float32),
        pltpu.VMEM((_K, 64), jnp.float32),
        pltpu.VMEM((_K,), jnp.float32),
        pltpu.VMEM((_K,), jnp.float32),
        pltpu.VMEM((_K,), jnp.float32),
        pltpu.VMEM((_NP,), jnp.float32),
        pltpu.SemaphoreType.DMA,
    ],
    compiler_params=pltpu.CompilerParams(needs_layout_passes=False, use_tc_tiling_on_sc=False),
)


_TE2 = _EP // _NS
_K2 = 512
_NB2 = _TE2 // _K2
_NR = _NP // _NS
_NRC = 800


def _make_agg_body(ph):
    def _agg_body(src_hbm, dst_hbm, att_hbm, xwq_hbm, zrows_hbm, out_hbm,
                  sv, dv, attv, xjh, wbuf, obuf, shacc, sem):
        q = ph * _NC + lax.axis_index("c")
        sid = lax.axis_index("s")
        pltpu.sync_copy(zrows_hbm, shacc.at[pl.ds(sid * _NR, _NR)])
        plsc.subcore_barrier()
        base = sid * _TE2

        def blk(b, _):
            off = base + b * _K2
            pltpu.sync_copy(src_hbm.at[pl.ds(off, _K2)], sv)
            pltpu.sync_copy(dst_hbm.at[pl.ds(off, _K2)], dv)
            c1 = pltpu.async_copy(att_hbm.at[pl.ds(off, _K2)], attv, sem)
            c2 = pltpu.async_copy(xwq_hbm.at[q].at[sv], xjh, sem)
            c1.wait(); c2.wait()

            def edge(i, _):
                for u in range(4):
                    e = i * 4 + u
                    a = plsc.load_gather(attv, [jnp.full((_L,), e, jnp.int32)])
                    wbuf[e, pl.ds(0, _L)] = xjh[e, pl.ds(0, _L)] * a
                return 0

            lax.fori_loop(0, _K2 // 4, edge, 0)
            pltpu.sync_copy(wbuf, shacc.at[dv], add=True)
            return 0

        lax.fori_loop(0, _NB2, blk, 0)
        plsc.subcore_barrier()
        for ch in range(_NR // _NRC):
            r0 = sid * _NR + ch * _NRC
            pltpu.sync_copy(shacc.at[pl.ds(r0, _NRC)], obuf)
            pltpu.sync_copy(obuf, out_hbm.at[lax.axis_index("c")].at[pl.ds(r0, _NRC)])
    return _agg_body


_agg_calls = [
    pl.kernel(
        _make_agg_body(ph),
        out_type=jax.ShapeDtypeStruct((_NC, _NP, _L), jnp.float32),
        mesh=_sc_mesh,
        scratch_types=[
            pltpu.VMEM((_K2,), jnp.int32),
            pltpu.VMEM((_K2,), jnp.int32),
            pltpu.VMEM((_K2,), jnp.float32),
            pltpu.VMEM((_K2, _L), jnp.float32),
            pltpu.VMEM((_K2, _L), jnp.float32),
            pltpu.VMEM((_NRC, _L), jnp.float32),
            pltpu.VMEM_SHARED((_NP, _L), jnp.float32),
            pltpu.SemaphoreType.DMA,
        ],
        compiler_params=pltpu.CompilerParams(needs_layout_passes=False, use_tc_tiling_on_sc=False),
    )
    for ph in range(2)
]


def _pad_edges(edge_index):
    src = jnp.pad(edge_index[0], (0, _EP - _E))
    dst = jnp.pad(edge_index[1], (0, _EP - _E))
    return src, dst


def _lrelu(t):
    return jnp.where(t >= 0, t, 0.01 * t)


def _combine_body(h_ref, w_ref, b_ref, xh_ref, o_ref):
    t = jnp.dot(h_ref[...], w_ref[...], preferred_element_type=jnp.float32)
    o_ref[...] = _lrelu(t + b_ref[...] + xh_ref[...])


def _combine(h, w, b, xh):
    n, d = h.shape
    grid = n // N_ROWS_BLK
    return pl.pallas_call(
        _combine_body,
        grid=(grid,),
        in_specs=[
            pl.BlockSpec((N_ROWS_BLK, d), lambda i: (i, 0)),
            pl.BlockSpec((d, d), lambda i: (0, 0)),
            pl.BlockSpec((1, d), lambda i: (0, 0)),
            pl.BlockSpec((N_ROWS_BLK, d), lambda i: (i, 0)),
        ],
        out_specs=pl.BlockSpec((N_ROWS_BLK, d), lambda i: (i, 0)),
        out_shape=jax.ShapeDtypeStruct((n, d), jnp.float32),
    )(h, w, b.reshape(1, d), xh)


def _l2norm(x):
    n = jnp.sqrt(jnp.sum(x * x, axis=-1, keepdims=True))
    return x / jnp.maximum(n, 1e-12)


def _graph_gat(x, edge_index, weight):
    n = x.shape[0]
    x = x @ weight
    dst = edge_index[1]
    src_p, dst_p = _pad_edges(edge_index)
    zeros_np = jnp.zeros((_NP,), jnp.float32)
    deg = _deg_call(src_p, dst_p, zeros_np).sum(axis=0)[:n]
    deg_inv_sqrt = jnp.where(deg > 0, deg, 1.0) ** -0.5
    rn = jnp.sqrt(jnp.sum(x * x, axis=-1))
    barr = jnp.maximum(rn * jnp.max(rn) - 40.0, 0.0)
    dis_p = jnp.pad(deg_inv_sqrt, (0, _NP - n))
    barr_p = jnp.pad(barr, (0, _NP - n))
    ex_p, dpart = _edge_call(src_p, dst_p, x, dis_p, barr_p, zeros_np)
    ex = ex_p[:_E]
    denom = dpart.sum(axis=0)[:n]
    att = ex / (jnp.take(denom, dst) + 1e-16)
    att_p = jnp.pad(att, (0, _EP - _E))
    xwq = jnp.stack([x[:, 0:16], x[:, 16:32], x[:, 32:48], x[:, 48:64]])
    zrows = jnp.zeros((_NR, _L), jnp.float32)
    o0 = _agg_calls[0](src_p, dst_p, att_p, xwq, zrows)
    o1 = _agg_calls[1](src_p, dst_p, att_p, xwq, zrows)
    out = jnp.concatenate([o0[0, :n], o0[1, :n], o1[0, :n], o1[1, :n]], axis=1)
    return _l2norm(out)


def kernel(id_embedding, edge_index, features, preference, mlp_w, mlp_b, conv1_w,
           lin1_w, lin1_b, g1_w, g1_b, conv2_w, lin2_w, lin2_b, g2_w, g2_b):
    temp_features = jnp.tanh(features @ mlp_w + mlp_b)
    x = jnp.concatenate([preference, temp_features], axis=0)
    x = _l2norm(x)
    h = _lrelu(_graph_gat(x, edge_index, conv1_w))
    x_hat = _lrelu(x @ lin1_w + lin1_b) + id_embedding
    x1 = _combine(h, g1_w, g1_b, x_hat)
    h = _lrelu(_graph_gat(x1, edge_index, conv2_w))
    x_hat = _lrelu(x1 @ lin2_w + lin2_b) + id_embedding
    x2 = _combine(h, g2_w, g2_b, x_hat)
    return jnp.concatenate([x1, x2], axis=1)


# speedup vs baseline: 6.2429x; 2.5754x over previous
import functools
import jax
import jax.numpy as jnp
from jax import lax
from jax.experimental import pallas as pl
from jax.experimental.pallas import tpu as pltpu
from jax.experimental.pallas import tpu_sc as plsc

N_ROWS_BLK = 1000

_NC = 2
_NS = 16
_NW = _NC * _NS
_L = 16

_N = 50000
_NP = 51200
_E = 800000
_EP = 802816
_TE = _EP // _NW
_K = 512
_NB = _TE // _K
_G = _K // _L

_sc_mesh = plsc.VectorSubcoreMesh(
    core_axis_name="c", subcore_axis_name="s", num_cores=_NC, num_subcores=_NS)


def _wid():
    return lax.axis_index("s") * _NC + lax.axis_index("c")


def _deg_body(src_hbm, dst_hbm, zeros_hbm, out_hbm, sv, dv, acc, sem):
    wid = _wid()
    pltpu.sync_copy(zeros_hbm, acc)
    base = wid * _TE

    def blk(b, _):
        off = base + b * _K
        pltpu.sync_copy(src_hbm.at[pl.ds(off, _K)], sv)
        pltpu.sync_copy(dst_hbm.at[pl.ds(off, _K)], dv)

        def grp(g, _):
            s16 = sv[pl.ds(g * _L, _L)]
            d16 = dv[pl.ds(g * _L, _L)]
            m = s16 != d16
            plsc.addupdate_scatter(acc, [s16], jnp.ones((_L,), jnp.float32), mask=m)
            return 0

        lax.fori_loop(0, _G, grp, 0)
        return 0

    lax.fori_loop(0, _NB, blk, 0)
    pltpu.sync_copy(acc, out_hbm.at[wid])


_deg_call = pl.kernel(
    _deg_body,
    out_type=jax.ShapeDtypeStruct((_NW, _NP), jnp.float32),
    mesh=_sc_mesh,
    scratch_types=[
        pltpu.VMEM((_K,), jnp.int32),
        pltpu.VMEM((_K,), jnp.int32),
        pltpu.VMEM((_NP,), jnp.float32),
        pltpu.SemaphoreType.DMA,
    ],
    compiler_params=pltpu.CompilerParams(needs_layout_passes=False, use_tc_tiling_on_sc=False),
)


def _edge_body(src_hbm, dst_hbm, xw_hbm, dis_hbm, barr_hbm, zeros_hbm,
               ex_hbm, dpart_hbm, sv, dv, xjv, xiv, disv, bv, exb, dacc, sem):
    wid = _wid()
    pltpu.sync_copy(zeros_hbm, dacc)
    base = wid * _TE
    lanes = lax.iota(jnp.int32, _L)

    def blk(b, _):
        off = base + b * _K
        pltpu.sync_copy(src_hbm.at[pl.ds(off, _K)], sv)
        pltpu.sync_copy(dst_hbm.at[pl.ds(off, _K)], dv)
        c1 = pltpu.async_copy(xw_hbm.at[sv], xjv, sem)
        c2 = pltpu.async_copy(xw_hbm.at[dv], xiv, sem)
        c3 = pltpu.async_copy(dis_hbm.at[sv], disv, sem)
        c4 = pltpu.async_copy(barr_hbm.at[dv], bv, sem)
        c1.wait(); c2.wait(); c3.wait(); c4.wait()

        def grp(g, _):
            e0 = g * _L
            el = lanes + e0
            s16 = sv[pl.ds(e0, _L)]
            d16 = dv[pl.ds(e0, _L)]
            m = s16 != d16
            acc = jnp.zeros((_L,), jnp.float32)
            for dcol in range(64):
                cc = jnp.full((_L,), dcol, jnp.int32)
                a = plsc.load_gather(xiv, [el, cc])
                bb = plsc.load_gather(xjv, [el, cc])
                acc = acc + a * bb
            dis16 = disv[pl.ds(e0, _L)]
            b16 = bv[pl.ds(e0, _L)]
            gate = 1.0 / (1.0 + jnp.exp(-dis16 * acc))
            ex16 = jnp.exp(acc * gate - b16)
            ex16 = jnp.where(m, ex16, 0.0)
            exb[pl.ds(e0, _L)] = ex16
            plsc.addupdate_scatter(dacc, [d16], ex16, mask=m)
            return 0

        lax.fori_loop(0, _G, grp, 0)
        pltpu.sync_copy(exb, ex_hbm.at[pl.ds(off, _K)])
        return 0

    lax.fori_loop(0, _NB, blk, 0)
    pltpu.sync_copy(dacc, dpart_hbm.at[wid])


_edge_call = pl.kernel(
    _edge_body,
    out_type=(jax.ShapeDtypeStruct((_EP,), jnp.float32),
              jax.ShapeDtypeStruct((_NW, _NP), jnp.float32)),
    mesh=_sc_mesh,
    scratch_types=[
        pltpu.VMEM((_K,), jnp.int32),
        pltpu.VMEM((_K,), jnp.int32),
        pltpu.VMEM((_K, 64), jnp.float32),
        pltpu.VMEM((_K, 64), jnp.float32),
        pltpu.VMEM((_K,), jnp.float32),
        pltpu.VMEM((_K,), jnp.float32),
        pltpu.VMEM((_K,), jnp.float32),
        pltpu.VMEM((_NP,), jnp.float32),
        pltpu.SemaphoreType.DMA,
    ],
    compiler_params=pltpu.CompilerParams(needs_layout_passes=False, use_tc_tiling_on_sc=False),
)


_TE2 = _EP // _NS
_K2 = 512
_NB2 = _TE2 // _K2
_NR = _NP // _NS
_NRC = 800


def _make_agg_body(ph):
    def _agg_body(src_hbm, dst_hbm, ex_hbm, den_hbm, xwq_hbm, zrows_hbm, out_hbm,
                  sv, dv, attv, denv, xjh, wbuf, obuf, shacc, sem):
        q = ph * _NC + lax.axis_index("c")
        sid = lax.axis_index("s")
        pltpu.sync_copy(zrows_hbm, shacc.at[pl.ds(sid * _NR, _NR)])
        plsc.subcore_barrier()
        base = sid * _TE2

        def blk(b, _):
            off = base + b * _K2
            pltpu.sync_copy(src_hbm.at[pl.ds(off, _K2)], sv)
            pltpu.sync_copy(dst_hbm.at[pl.ds(off, _K2)], dv)
            c1 = pltpu.async_copy(ex_hbm.at[pl.ds(off, _K2)], attv, sem)
            c2 = pltpu.async_copy(xwq_hbm.at[q].at[sv], xjh, sem)
            c3 = pltpu.async_copy(den_hbm.at[dv], denv, sem)
            c1.wait(); c2.wait(); c3.wait()

            def attg(g, _):
                sl = pl.ds(g * _L, _L)
                attv[sl] = attv[sl] / (denv[sl] + 1e-16)
                return 0

            lax.fori_loop(0, _K2 // _L, attg, 0)

            def edge(i, _):
                for u in range(4):
                    e = i * 4 + u
                    a = plsc.load_gather(attv, [jnp.full((_L,), e, jnp.int32)])
                    wbuf[e, pl.ds(0, _L)] = xjh[e, pl.ds(0, _L)] * a
                return 0

            lax.fori_loop(0, _K2 // 4, edge, 0)
            pltpu.sync_copy(wbuf, shacc.at[dv], add=True)
            return 0

        lax.fori_loop(0, _NB2, blk, 0)
        plsc.subcore_barrier()
        for ch in range(_NR // _NRC):
            r0 = sid * _NR + ch * _NRC
            pltpu.sync_copy(shacc.at[pl.ds(r0, _NRC)], obuf)
            pltpu.sync_copy(obuf, out_hbm.at[lax.axis_index("c")].at[pl.ds(r0, _NRC)])
    return _agg_body


_agg_calls = [
    pl.kernel(
        _make_agg_body(ph),
        out_type=jax.ShapeDtypeStruct((_NC, _NP, _L), jnp.float32),
        mesh=_sc_mesh,
        scratch_types=[
            pltpu.VMEM((_K2,), jnp.int32),
            pltpu.VMEM((_K2,), jnp.int32),
            pltpu.VMEM((_K2,), jnp.float32),
            pltpu.VMEM((_K2,), jnp.float32),
            pltpu.VMEM((_K2, _L), jnp.float32),
            pltpu.VMEM((_K2, _L), jnp.float32),
            pltpu.VMEM((_NRC, _L), jnp.float32),
            pltpu.VMEM_SHARED((_NP, _L), jnp.float32),
            pltpu.SemaphoreType.DMA,
        ],
        compiler_params=pltpu.CompilerParams(needs_layout_passes=False, use_tc_tiling_on_sc=False),
    )
    for ph in range(2)
]


def _pad_edges(edge_index):
    src = jnp.pad(edge_index[0], (0, _EP - _E))
    dst = jnp.pad(edge_index[1], (0, _EP - _E))
    return src, dst


def _lrelu(t):
    return jnp.where(t >= 0, t, 0.01 * t)


def _combine_body(h_ref, w_ref, b_ref, xh_ref, o_ref):
    t = jnp.dot(h_ref[...], w_ref[...], preferred_element_type=jnp.float32)
    o_ref[...] = _lrelu(t + b_ref[...] + xh_ref[...])


def _combine(h, w, b, xh):
    n, d = h.shape
    grid = n // N_ROWS_BLK
    return pl.pallas_call(
        _combine_body,
        grid=(grid,),
        in_specs=[
            pl.BlockSpec((N_ROWS_BLK, d), lambda i: (i, 0)),
            pl.BlockSpec((d, d), lambda i: (0, 0)),
            pl.BlockSpec((1, d), lambda i: (0, 0)),
            pl.BlockSpec((N_ROWS_BLK, d), lambda i: (i, 0)),
        ],
        out_specs=pl.BlockSpec((N_ROWS_BLK, d), lambda i: (i, 0)),
        out_shape=jax.ShapeDtypeStruct((n, d), jnp.float32),
    )(h, w, b.reshape(1, d), xh)


def _l2norm(x):
    n = jnp.sqrt(jnp.sum(x * x, axis=-1, keepdims=True))
    return x / jnp.maximum(n, 1e-12)


def _graph_gat(x, src_p, dst_p, dis_p, weight):
    n = x.shape[0]
    x = x @ weight
    zeros_np = jnp.zeros((_NP,), jnp.float32)
    rn = jnp.sqrt(jnp.sum(x * x, axis=-1))
    barr = jnp.maximum(rn * jnp.max(rn) - 40.0, 0.0)
    barr_p = jnp.pad(barr, (0, _NP - n))
    ex_p, dpart = _edge_call(src_p, dst_p, x, dis_p, barr_p, zeros_np)
    den_p = jnp.pad(dpart.sum(axis=0)[:n], (0, _NP - n))
    xwq = jnp.stack([x[:, 0:16], x[:, 16:32], x[:, 32:48], x[:, 48:64]])
    zrows = jnp.zeros((_NR, _L), jnp.float32)
    o0 = _agg_calls[0](src_p, dst_p, ex_p, den_p, xwq, zrows)
    o1 = _agg_calls[1](src_p, dst_p, ex_p, den_p, xwq, zrows)
    out = jnp.concatenate([o0[0, :n], o0[1, :n], o1[0, :n], o1[1, :n]], axis=1)
    return _l2norm(out)


def kernel(id_embedding, edge_index, features, preference, mlp_w, mlp_b, conv1_w,
           lin1_w, lin1_b, g1_w, g1_b, conv2_w, lin2_w, lin2_b, g2_w, g2_b):
    temp_features = jnp.tanh(features @ mlp_w + mlp_b)
    x = jnp.concatenate([preference, temp_features], axis=0)
    x = _l2norm(x)
    n = x.shape[0]
    src_p, dst_p = _pad_edges(edge_index)
    zeros_np = jnp.zeros((_NP,), jnp.float32)
    deg = _deg_call(src_p, dst_p, zeros_np).sum(axis=0)[:n]
    dis_p = jnp.pad(jnp.where(deg > 0, deg, 1.0) ** -0.5, (0, _NP - n))
    h = _lrelu(_graph_gat(x, src_p, dst_p, dis_p, conv1_w))
    x_hat = _lrelu(x @ lin1_w + lin1_b) + id_embedding
    x1 = _combine(h, g1_w, g1_b, x_hat)
    h = _lrelu(_graph_gat(x1, src_p, dst_p, dis_p, conv2_w))
    x_hat = _lrelu(x1 @ lin2_w + lin2_b) + id_embedding
    x2 = _combine(h, g2_w, g2_b, x_hat)
    return jnp.concatenate([x1, x2], axis=1)


# speedup vs baseline: 6.8106x; 1.0909x over previous
import functools
import jax
import jax.numpy as jnp
from jax import lax
from jax.experimental import pallas as pl
from jax.experimental.pallas import tpu as pltpu
from jax.experimental.pallas import tpu_sc as plsc

N_ROWS_BLK = 1000

_NC = 2
_NS = 16
_NW = _NC * _NS
_L = 16

_N = 50000
_NP = 51200
_E = 800000
_EP = 802816
_TE = _EP // _NW
_K = 512
_NB = _TE // _K
_G = _K // _L

_sc_mesh = plsc.VectorSubcoreMesh(
    core_axis_name="c", subcore_axis_name="s", num_cores=_NC, num_subcores=_NS)


def _wid():
    return lax.axis_index("s") * _NC + lax.axis_index("c")


def _deg_body(src_hbm, dst_hbm, zeros_hbm, out_hbm, sv, dv, acc, sem):
    wid = _wid()
    pltpu.sync_copy(zeros_hbm, acc)
    base = wid * _TE

    def blk(b, _):
        off = base + b * _K
        pltpu.sync_copy(src_hbm.at[pl.ds(off, _K)], sv)
        pltpu.sync_copy(dst_hbm.at[pl.ds(off, _K)], dv)

        def grp(g, _):
            s16 = sv[pl.ds(g * _L, _L)]
            d16 = dv[pl.ds(g * _L, _L)]
            m = s16 != d16
            plsc.addupdate_scatter(acc, [s16], jnp.ones((_L,), jnp.float32), mask=m)
            return 0

        lax.fori_loop(0, _G, grp, 0)
        return 0

    lax.fori_loop(0, _NB, blk, 0)
    pltpu.sync_copy(acc, out_hbm.at[wid])


_deg_call = pl.kernel(
    _deg_body,
    out_type=jax.ShapeDtypeStruct((_NW, _NP), jnp.float32),
    mesh=_sc_mesh,
    scratch_types=[
        pltpu.VMEM((_K,), jnp.int32),
        pltpu.VMEM((_K,), jnp.int32),
        pltpu.VMEM((_NP,), jnp.float32),
        pltpu.SemaphoreType.DMA,
    ],
    compiler_params=pltpu.CompilerParams(needs_layout_passes=False, use_tc_tiling_on_sc=False),
)


_KE = 256
_GE = _KE // _L
_NBE = _TE // _KE


def _edge_body(src_hbm, dst_hbm, xw_hbm, dis_hbm, barr_hbm, zeros_hbm,
               ex_hbm, dpart_hbm,
               sv0, dv0, sv1, dv1, xj0, xi0, xj1, xi1, ds0, bv0, ds1, bv1,
               exb, dacc, semi0, semi1, semr0, semr1):
    svs, dvs = [sv0, sv1], [dv0, dv1]
    xjs, xis = [xj0, xj1], [xi0, xi1]
    dss, bvs = [ds0, ds1], [bv0, bv1]
    semi, semr = [semi0, semi1], [semr0, semr1]
    wid = _wid()
    pltpu.sync_copy(zeros_hbm, dacc)
    base = wid * _TE
    lanes = lax.iota(jnp.int32, _L)

    def off_of(b):
        return base + jnp.minimum(b, _NBE - 1) * _KE

    def issue_idx(b, s):
        off = off_of(b)
        pltpu.async_copy(src_hbm.at[pl.ds(off, _KE)], svs[s], semi[s])
        pltpu.async_copy(dst_hbm.at[pl.ds(off, _KE)], dvs[s], semi[s])

    def wait_idx(s):
        pltpu.make_async_copy(src_hbm.at[pl.ds(0, _KE)], svs[s], semi[s]).wait()
        pltpu.make_async_copy(dst_hbm.at[pl.ds(0, _KE)], dvs[s], semi[s]).wait()

    def issue_rows(s):
        pltpu.async_copy(xw_hbm.at[svs[s]], xjs[s], semr[s])
        pltpu.async_copy(xw_hbm.at[dvs[s]], xis[s], semr[s])
        pltpu.async_copy(dis_hbm.at[svs[s]], dss[s], semr[s])
        pltpu.async_copy(barr_hbm.at[dvs[s]], bvs[s], semr[s])

    def wait_rows(s):
        pltpu.make_async_copy(xw_hbm.at[pl.ds(0, _KE)], xjs[s], semr[s]).wait()
        pltpu.make_async_copy(xw_hbm.at[pl.ds(0, _KE)], xis[s], semr[s]).wait()
        pltpu.make_async_copy(dis_hbm.at[pl.ds(0, _KE)], dss[s], semr[s]).wait()
        pltpu.make_async_copy(barr_hbm.at[pl.ds(0, _KE)], bvs[s], semr[s]).wait()

    def compute(b, s):
        sv, dv, xjv, xiv, disv, bv = svs[s], dvs[s], xjs[s], xis[s], dss[s], bvs[s]

        def grp(g, _):
            e0 = g * _L
            el = lanes + e0
            s16 = sv[pl.ds(e0, _L)]
            d16 = dv[pl.ds(e0, _L)]
            m = s16 != d16
            acc = jnp.zeros((_L,), jnp.float32)
            for dcol in range(64):
                cc = jnp.full((_L,), dcol, jnp.int32)
                a = plsc.load_gather(xiv, [el, cc])
                bb = plsc.load_gather(xjv, [el, cc])
                acc = acc + a * bb
            dis16 = disv[pl.ds(e0, _L)]
            b16 = bv[pl.ds(e0, _L)]
            gate = 1.0 / (1.0 + jnp.exp(-dis16 * acc))
            ex16 = jnp.exp(acc * gate - b16)
            ex16 = jnp.where(m, ex16, 0.0)
            exb[pl.ds(e0, _L)] = ex16
            plsc.addupdate_scatter(dacc, [d16], ex16, mask=m)
            return 0

        lax.fori_loop(0, _GE, grp, 0)
        pltpu.sync_copy(exb, ex_hbm.at[pl.ds(off_of(b), _KE)])

    issue_idx(0, 0)
    wait_idx(0)
    issue_rows(0)
    issue_idx(1, 1)

    def pair(p, _):
        b0 = 2 * p
        b1 = b0 + 1
        wait_idx(1)
        issue_rows(1)
        wait_rows(0)
        compute(b0, 0)
        issue_idx(b0 + 2, 0)
        wait_idx(0)
        issue_rows(0)
        wait_rows(1)
        compute(b1, 1)
        issue_idx(b1 + 2, 1)
        return 0

    lax.fori_loop(0, _NBE // 2, pair, 0)
    wait_rows(0)
    wait_idx(1)
    pltpu.sync_copy(dacc, dpart_hbm.at[wid])


_edge_call = pl.kernel(
    _edge_body,
    out_type=(jax.ShapeDtypeStruct((_EP,), jnp.float32),
              jax.ShapeDtypeStruct((_NW, _NP), jnp.float32)),
    mesh=_sc_mesh,
    scratch_types=(
        [pltpu.VMEM((_KE,), jnp.int32)] * 4
        + [pltpu.VMEM((_KE, 64), jnp.float32)] * 4
        + [pltpu.VMEM((_KE,), jnp.float32)] * 4
        + [pltpu.VMEM((_KE,), jnp.float32), pltpu.VMEM((_NP,), jnp.float32)]
        + [pltpu.SemaphoreType.DMA] * 4
    ),
    compiler_params=pltpu.CompilerParams(needs_layout_passes=False, use_tc_tiling_on_sc=False),
)


_TE2 = _EP // _NS
_K2 = 512
_NB2 = _TE2 // _K2
_NR = _NP // _NS
_NRC = 800


def _make_agg_body(ph):
    def _agg_body(src_hbm, dst_hbm, ex_hbm, den_hbm, xwq_hbm, zrows_hbm, out_hbm,
                  sv, dv, attv, denv, xjh, wbuf, obuf, shacc, sem):
        q = ph * _NC + lax.axis_index("c")
        sid = lax.axis_index("s")
        pltpu.sync_copy(zrows_hbm, shacc.at[pl.ds(sid * _NR, _NR)])
        plsc.subcore_barrier()
        base = sid * _TE2

        def blk(b, _):
            off = base + b * _K2
            pltpu.sync_copy(src_hbm.at[pl.ds(off, _K2)], sv)
            pltpu.sync_copy(dst_hbm.at[pl.ds(off, _K2)], dv)
            c1 = pltpu.async_copy(ex_hbm.at[pl.ds(off, _K2)], attv, sem)
            c2 = pltpu.async_copy(xwq_hbm.at[q].at[sv], xjh, sem)
            c3 = pltpu.async_copy(den_hbm.at[dv], denv, sem)
            c1.wait(); c2.wait(); c3.wait()

            def attg(g, _):
                sl = pl.ds(g * _L, _L)
                attv[sl] = attv[sl] / (denv[sl] + 1e-16)
                return 0

            lax.fori_loop(0, _K2 // _L, attg, 0)

            def edge(i, _):
                for u in range(4):
                    e = i * 4 + u
                    a = plsc.load_gather(attv, [jnp.full((_L,), e, jnp.int32)])
                    wbuf[e, pl.ds(0, _L)] = xjh[e, pl.ds(0, _L)] * a
                return 0

            lax.fori_loop(0, _K2 // 4, edge, 0)
            pltpu.sync_copy(wbuf, shacc.at[dv], add=True)
            return 0

        lax.fori_loop(0, _NB2, blk, 0)
        plsc.subcore_barrier()
        for ch in range(_NR // _NRC):
            r0 = sid * _NR + ch * _NRC
            pltpu.sync_copy(shacc.at[pl.ds(r0, _NRC)], obuf)
            pltpu.sync_copy(obuf, out_hbm.at[lax.axis_index("c")].at[pl.ds(r0, _NRC)])
    return _agg_body


_agg_calls = [
    pl.kernel(
        _make_agg_body(ph),
        out_type=jax.ShapeDtypeStruct((_NC, _NP, _L), jnp.float32),
        mesh=_sc_mesh,
        scratch_types=[
            pltpu.VMEM((_K2,), jnp.int32),
            pltpu.VMEM((_K2,), jnp.int32),
            pltpu.VMEM((_K2,), jnp.float32),
            pltpu.VMEM((_K2,), jnp.float32),
            pltpu.VMEM((_K2, _L), jnp.float32),
            pltpu.VMEM((_K2, _L), jnp.float32),
            pltpu.VMEM((_NRC, _L), jnp.float32),
            pltpu.VMEM_SHARED((_NP, _L), jnp.float32),
            pltpu.SemaphoreType.DMA,
        ],
        compiler_params=pltpu.CompilerParams(needs_layout_passes=False, use_tc_tiling_on_sc=False),
    )
    for ph in range(2)
]


def _pad_edges(edge_index):
    src = jnp.pad(edge_index[0], (0, _EP - _E))
    dst = jnp.pad(edge_index[1], (0, _EP - _E))
    return src, dst


def _lrelu(t):
    return jnp.where(t >= 0, t, 0.01 * t)


def _combine_body(h_ref, w_ref, b_ref, xh_ref, o_ref):
    t = jnp.dot(h_ref[...], w_ref[...], preferred_element_type=jnp.float32)
    o_ref[...] = _lrelu(t + b_ref[...] + xh_ref[...])


def _combine(h, w, b, xh):
    n, d = h.shape
    grid = n // N_ROWS_BLK
    return pl.pallas_call(
        _combine_body,
        grid=(grid,),
        in_specs=[
            pl.BlockSpec((N_ROWS_BLK, d), lambda i: (i, 0)),
            pl.BlockSpec((d, d), lambda i: (0, 0)),
            pl.BlockSpec((1, d), lambda i: (0, 0)),
            pl.BlockSpec((N_ROWS_BLK, d), lambda i: (i, 0)),
        ],
        out_specs=pl.BlockSpec((N_ROWS_BLK, d), lambda i: (i, 0)),
        out_shape=jax.ShapeDtypeStruct((n, d), jnp.float32),
    )(h, w, b.reshape(1, d), xh)


def _l2norm(x):
    n = jnp.sqrt(jnp.sum(x * x, axis=-1, keepdims=True))
    return x / jnp.maximum(n, 1e-12)


def _graph_gat(x, src_p, dst_p, dis_p, weight):
    n = x.shape[0]
    x = x @ weight
    zeros_np = jnp.zeros((_NP,), jnp.float32)
    rn = jnp.sqrt(jnp.sum(x * x, axis=-1))
    barr = jnp.maximum(rn * jnp.max(rn) - 40.0, 0.0)
    barr_p = jnp.pad(barr, (0, _NP - n))
    ex_p, dpart = _edge_call(src_p, dst_p, x, dis_p, barr_p, zeros_np)
    den_p = jnp.pad(dpart.sum(axis=0)[:n], (0, _NP - n))
    xwq = jnp.stack([x[:, 0:16], x[:, 16:32], x[:, 32:48], x[:, 48:64]])
    zrows = jnp.zeros((_NR, _L), jnp.float32)
    o0 = _agg_calls[0](src_p, dst_p, ex_p, den_p, xwq, zrows)
    o1 = _agg_calls[1](src_p, dst_p, ex_p, den_p, xwq, zrows)
    out = jnp.concatenate([o0[0, :n], o0[1, :n], o1[0, :n], o1[1, :n]], axis=1)
    return _l2norm(out)


def kernel(id_embedding, edge_index, features, preference, mlp_w, mlp_b, conv1_w,
           lin1_w, lin1_b, g1_w, g1_b, conv2_w, lin2_w, lin2_b, g2_w, g2_b):
    temp_features = jnp.tanh(features @ mlp_w + mlp_b)
    x = jnp.concatenate([preference, temp_features], axis=0)
    x = _l2norm(x)
    n = x.shape[0]
    src_p, dst_p = _pad_edges(edge_index)
    zeros_np = jnp.zeros((_NP,), jnp.float32)
    deg = _deg_call(src_p, dst_p, zeros_np).sum(axis=0)[:n]
    dis_p = jnp.pad(jnp.where(deg > 0, deg, 1.0) ** -0.5, (0, _NP - n))
    h = _lrelu(_graph_gat(x, src_p, dst_p, dis_p, conv1_w))
    x_hat = _lrelu(x @ lin1_w + lin1_b) + id_embedding
    x1 = _combine(h, g1_w, g1_b, x_hat)
    h = _lrelu(_graph_gat(x1, src_p, dst_p, dis_p, conv2_w))
    x_hat = _lrelu(x1 @ lin2_w + lin2_b) + id_embedding
    x2 = _combine(h, g2_w, g2_b, x_hat)
    return jnp.concatenate([x1, x2], axis=1)
